# Initial kernel scaffold; baseline (speedup 1.0000x reference)
import jax
import jax.numpy as jnp
from jax.experimental import pallas as pl


def kernel(x, edge_index, edge_attr, W, b):
    raise NotImplementedError("write your pallas kernel here")


# speedup vs baseline: 4.6540x; 4.6540x over previous
import functools

import jax
import jax.numpy as jnp
from jax import lax
from jax.experimental import pallas as pl
from jax.experimental.pallas import tpu as pltpu
from jax.experimental.pallas import tpu_sc as plsc

NC = 2
NS = 16
NW = NC * NS
L = 16
B = 80
SUP = 4000
SPAD = 4096
N_REAL = 10000
N_PAD = 10240
RPT = N_PAD // NW

_sc_params = pltpu.CompilerParams(needs_layout_passes=False)


def _iota16():
    return lax.broadcasted_iota(jnp.int32, (L,), 0)


def _bcast(v, i):
    return v.at[jnp.full((L,), i, jnp.int32)].get(mode="promise_in_bounds")


def _rsqrt16(d):
    i = lax.bitcast_convert_type(d, jnp.int32)
    i = jnp.int32(0x5F3759DF) - lax.shift_right_arithmetic(i, 1)
    g = lax.bitcast_convert_type(i, jnp.float32)
    for _ in range(3):
        g = g * (1.5 - 0.5 * d * g * g)
    return g


def _tc_linear(x_p, w, deg0, deg1):
    n_pad, cin = x_p.shape
    cout = w.shape[1]
    blk = 1024
    d0 = deg0.reshape(n_pad, 1)
    d1 = deg1.reshape(n_pad, 1)

    def body(x_ref, w_ref, d0_ref, d1_ref, o_ref, dt_ref):
        i = pl.program_id(0)
        rows = i * blk + lax.broadcasted_iota(jnp.int32, (blk, 1), 0)
        o_ref[...] = jnp.dot(x_ref[...], w_ref[...],
                             preferred_element_type=jnp.float32)
        dt_ref[...] = (d0_ref[...] + d1_ref[...]
                       + jnp.where(rows < N_REAL, 1.0, 0.0))

    return pl.pallas_call(
        body,
        grid=(n_pad // blk,),
        in_specs=[
            pl.BlockSpec((blk, cin), lambda i: (i, 0)),
            pl.BlockSpec((cin, cout), lambda i: (0, 0)),
            pl.BlockSpec((blk, 1), lambda i: (i, 0)),
            pl.BlockSpec((blk, 1), lambda i: (i, 0)),
        ],
        out_specs=[
            pl.BlockSpec((blk, cout), lambda i: (i, 0)),
            pl.BlockSpec((blk, 1), lambda i: (i, 0)),
        ],
        out_shape=[
            jax.ShapeDtypeStruct((n_pad, cout), jnp.float32),
            jax.ShapeDtypeStruct((n_pad, 1), jnp.float32),
        ],
    )(x_p, w, d0, d1)


def _tc_combine(acc_p, xlin_p, degt2, b):
    n_pad, cout = acc_p.shape
    blk = 1024
    b2 = b.reshape(1, cout)

    def body(acc_ref, xl_ref, dt_ref, b_ref, o_ref):
        i = pl.program_id(0)
        rows = i * blk + lax.broadcasted_iota(jnp.int32, (blk, 1), 0)
        dt = dt_ref[...]
        dis = jnp.where(rows < N_REAL,
                        lax.rsqrt(jnp.maximum(dt, 1e-12)), 0.0)
        o_ref[...] = dis * acc_ref[...] + (dis * dis) * xl_ref[...] + b_ref[...]

    return pl.pallas_call(
        body,
        grid=(n_pad // blk,),
        in_specs=[
            pl.BlockSpec((blk, cout), lambda i: (i, 0)),
            pl.BlockSpec((blk, cout), lambda i: (i, 0)),
            pl.BlockSpec((blk, 1), lambda i: (i, 0)),
            pl.BlockSpec((1, cout), lambda i: (0, 0)),
        ],
        out_specs=pl.BlockSpec((blk, cout), lambda i: (i, 0)),
        out_shape=jax.ShapeDtypeStruct((n_pad, cout), jnp.float32),
    )(acc_p, xlin_p, degt2, b2)


def _make_sc_deg(e):
    ept = e // NW
    nv = (ept + L - 1) // L
    spad = nv * L
    mesh = plsc.VectorSubcoreMesh(core_axis_name="c", subcore_axis_name="s")

    @functools.partial(
        pl.kernel,
        out_type=(
            jax.ShapeDtypeStruct((N_PAD,), jnp.float32),
            jax.ShapeDtypeStruct((N_PAD,), jnp.float32),
        ),
        mesh=mesh,
        compiler_params=_sc_params,
        scratch_types=(
            pltpu.VMEM((spad,), jnp.int32),
            pltpu.VMEM((spad,), jnp.float32),
            pltpu.VMEM((N_PAD,), jnp.float32),
            pltpu.VMEM((N_PAD // NS,), jnp.float32),
            pltpu.VMEM((N_PAD // NS,), jnp.float32),
            pltpu.VMEM_SHARED((NS, N_PAD), jnp.float32),
        ),
    )
    def deg(col_hbm, w_hbm, p0_out, p1_out, colb, wb, hist, racc, rtmp, part):
        c = lax.axis_index("c")
        s = lax.axis_index("s")
        wid = c * NS + s
        drs = N_PAD // NS
        zeros = jnp.zeros((L,), jnp.float32)

        def zh(k, _):
            hist[pl.ds(k * L, L)] = zeros
            return 0
        lax.fori_loop(0, N_PAD // L, zh, 0)
        colb[pl.ds(spad - L, L)] = jnp.zeros((L,), jnp.int32)
        wb[pl.ds(spad - L, L)] = zeros
        pltpu.sync_copy(col_hbm.at[pl.ds(wid * ept, ept)], colb.at[pl.ds(0, ept)])
        pltpu.sync_copy(w_hbm.at[pl.ds(wid * ept, ept)], wb.at[pl.ds(0, ept)])

        lane0 = _iota16() == 0

        def hloop(k, _):
            cv = colb[pl.ds(k * L, L)]
            wv = wb[pl.ds(k * L, L)]
            for i in range(L):
                plsc.addupdate_scatter(hist, [_bcast(cv, i)], _bcast(wv, i),
                                       mask=lane0)
            return 0
        lax.fori_loop(0, nv, hloop, 0)

        pltpu.sync_copy(hist, part.at[s])
        plsc.subcore_barrier()

        pltpu.sync_copy(part.at[0, pl.ds(s * drs, drs)], racc)

        def rloop(t, _):
            pltpu.sync_copy(part.at[t, pl.ds(s * drs, drs)], rtmp)

            def addv(k, _):
                racc[pl.ds(k * L, L)] = (racc[pl.ds(k * L, L)]
                                         + rtmp[pl.ds(k * L, L)])
                return 0
            return lax.fori_loop(0, drs // L, addv, 0)
        lax.fori_loop(1, NS, rloop, 0)

        @pl.when(c == 0)
        def _():
            pltpu.sync_copy(racc, p0_out.at[pl.ds(s * drs, drs)])

        @pl.when(c == 1)
        def _():
            pltpu.sync_copy(racc, p1_out.at[pl.ds(s * drs, drs)])

    return deg


def _make_sc_agg(e, cout):
    nsup = e // SUP
    mesh = plsc.VectorSubcoreMesh(core_axis_name="c", subcore_axis_name="s")

    @functools.partial(
        pl.kernel,
        out_type=jax.ShapeDtypeStruct((N_PAD, cout), jnp.float32),
        mesh=mesh,
        compiler_params=_sc_params,
        scratch_types=(
            pltpu.VMEM((RPT, cout), jnp.float32),
            pltpu.VMEM((SPAD,), jnp.int32),
            pltpu.VMEM((SPAD,), jnp.int32),
            pltpu.VMEM((SPAD,), jnp.float32),
            pltpu.VMEM((N_PAD,), jnp.float32),
            pltpu.VMEM((B, cout), jnp.float32),
            pltpu.SemaphoreType.DMA,
        ),
    )
    def agg(xlin_hbm, row_hbm, col_hbm, w_hbm, degt_hbm, acc_out,
            acc, srow, scol, sw, dis, gbuf, sem):
        c = lax.axis_index("c")
        s = lax.axis_index("s")
        wid = c * NS + s
        lo = wid * RPT
        zeros = jnp.zeros((L,), jnp.float32)
        izeros = jnp.zeros((L,), jnp.int32)
        iota = _iota16()

        def zacc(r, _):
            for j in range(cout // L):
                acc[r, pl.ds(j * L, L)] = zeros
            return 0
        lax.fori_loop(0, RPT, zacc, 0)
        for t in range((SPAD - SUP) // L):
            srow[pl.ds(SUP + t * L, L)] = izeros
            scol[pl.ds(SUP + t * L, L)] = izeros
            sw[pl.ds(SUP + t * L, L)] = zeros

        pltpu.sync_copy(degt_hbm, dis)

        def mk_dis(k, _):
            dis[pl.ds(k * L, L)] = _rsqrt16(dis[pl.ds(k * L, L)])
            return 0
        lax.fori_loop(0, N_PAD // L, mk_dis, 0)

        def pc_super(sp, _):
            off = sp * SUP
            pltpu.sync_copy(row_hbm.at[pl.ds(off, SUP)], srow.at[pl.ds(0, SUP)])
            pltpu.sync_copy(col_hbm.at[pl.ds(off, SUP)], scol.at[pl.ds(0, SUP)])
            pltpu.sync_copy(w_hbm.at[pl.ds(off, SUP)], sw.at[pl.ds(0, SUP)])

            def scan(k, cnt):
                rv = srow[pl.ds(k * L, L)]
                cv = scol[pl.ds(k * L, L)]
                wv = sw[pl.ds(k * L, L)]
                t = cv - lo
                m = t.astype(jnp.uint32) < jnp.uint32(RPT)
                plsc.store_compressed(srow.at[pl.ds(cnt, L)], rv, mask=m)
                plsc.store_compressed(scol.at[pl.ds(cnt, L)], t, mask=m)
                plsc.store_compressed(sw.at[pl.ds(cnt, L)], wv, mask=m)
                return cnt + jnp.sum(m.astype(jnp.int32))
            cnt = lax.fori_loop(0, SUP // L, scan, jnp.int32(0))

            base0 = (cnt // L) * L

            def ztail(t, _):
                bs = base0 + t * L
                idx = bs + iota
                live = idx < cnt
                scol[pl.ds(bs, L)] = jnp.where(live, scol[pl.ds(bs, L)], 0)
                sw[pl.ds(bs, L)] = jnp.where(live, sw[pl.ds(bs, L)], 0.0)
                return 0
            lax.fori_loop(0, (B + 2 * L) // L, ztail, 0)

            nb = (cnt + (B - 1)) // B

            def batch(bi, _):
                pltpu.async_copy(
                    xlin_hbm.at[srow.at[pl.ds(bi * B, B)]], gbuf, sem).wait()

                def rblk(rb, _):
                    o = bi * B + rb * L
                    dv = scol[pl.ds(o, L)]
                    wv = sw[pl.ds(o, L)]
                    rv = srow[pl.ds(o, L)]
                    sv = wv * plsc.load_gather(dis, [rv])
                    for i in range(L):
                        db = _bcast(dv, i)
                        sb = _bcast(sv, i)
                        r = rb * L + i
                        for j in range(cout // L):
                            cidx = jnp.int32(j * L) + iota
                            plsc.addupdate_scatter(
                                acc, [db, cidx],
                                gbuf[r, pl.ds(j * L, L)] * sb)
                    return 0
                return lax.fori_loop(0, B // L, rblk, 0)
            lax.fori_loop(0, nb, batch, 0)
            return 0
        lax.fori_loop(0, nsup, pc_super, 0)

        for z in range(RPT // B):
            pltpu.sync_copy(acc.at[pl.ds(z * B, B)],
                            acc_out.at[pl.ds(lo + z * B, B)])

    return agg


def kernel(x, edge_index, edge_attr, W, b):
    n, _ = x.shape
    e = edge_index.shape[1]
    cout = W.shape[1]

    x_p = jnp.pad(x, ((0, N_PAD - n), (0, 0)))
    row = edge_index[0]
    col = edge_index[1]

    deg0, deg1 = _make_sc_deg(e)(col, edge_attr)
    xlin_p, degt2 = _tc_linear(x_p, W, deg0, deg1)
    acc_p = _make_sc_agg(e, cout)(xlin_p, row, col, edge_attr,
                                  degt2.reshape(N_PAD))
    out_p = _tc_combine(acc_p, xlin_p, degt2, b)
    return out_p[:n]
